# Initial kernel scaffold; baseline (speedup 1.0000x reference)
import jax
import jax.numpy as jnp
from jax.experimental import pallas as pl


def kernel(node_feats, positions, edge_attr, eW1, eb1, eW2, eb2, fW1, fb1, fW2, fb2, pW1, pb1, pW2, edge_index):
    raise NotImplementedError("write your pallas kernel here")


# speedup vs baseline: 2.8920x; 2.8920x over previous
import functools

import jax
import jax.numpy as jnp
from jax import lax
from jax.experimental import pallas as pl
from jax.experimental.pallas import tpu as pltpu
from jax.experimental.pallas import tpu_sc as plsc

NC = 2
NS = 16
NW = NC * NS
CH = 80


def _prep_body(nf_ref, ws_ref, wd_ref, a_ref, b_ref):
    nf = nf_ref[...]
    a_ref[...] = jnp.dot(nf, ws_ref[...], preferred_element_type=jnp.float32)
    b_ref[...] = jnp.dot(nf, wd_ref[...], preferred_element_type=jnp.float32)


def _gather_sc(a_hbm, b_hbm, p_hbm, src_hbm, dst_hbm, gsum_hbm, rel_hbm,
               idx_s, idx_d, arows, brows, ps, pd, relv, sA, sB, sC, sD):
    E = src_hbm.shape[0]
    per = E // NW
    wid = lax.axis_index("s") * NC + lax.axis_index("c")

    @pl.loop(0, per // CH)
    def _chunk(t):
        base = wid * per + t * CH
        pltpu.sync_copy(src_hbm.at[pl.ds(base, CH)], idx_s)
        pltpu.sync_copy(dst_hbm.at[pl.ds(base, CH)], idx_d)
        cA = pltpu.async_copy(a_hbm.at[idx_s], arows, sA)
        cB = pltpu.async_copy(b_hbm.at[idx_d], brows, sB)
        cP = pltpu.async_copy(p_hbm.at[idx_s], ps, sC)
        cQ = pltpu.async_copy(p_hbm.at[idx_d], pd, sD)
        cA.wait(); cB.wait(); cP.wait(); cQ.wait()

        @pl.loop(0, CH)
        def _row(i):
            for j in range(8):
                sl = pl.ds(j * 16, 16)
                arows[i, sl] = arows[i, sl] + brows[i, sl]
            relv[i, :] = ps[i, :] - pd[i, :]

        pltpu.sync_copy(arows, gsum_hbm.at[pl.ds(base, CH)])
        pltpu.sync_copy(relv, rel_hbm.at[pl.ds(base, CH)])


def _edge_body(gsum_ref, rel_ref, ea_ref, eb1_ref, wr_ref, we_ref, ew2_ref,
               eb2_ref, pw1_ref, pb1_ref, pw2_ref, mij_ref, wv_ref):
    rel = rel_ref[...]
    dist = jnp.sum(rel * rel, axis=1, keepdims=True)
    r = jnp.sqrt(dist)
    reln = rel / (r + 0.1)
    pre = (gsum_ref[...] + eb1_ref[...] + dist * wr_ref[...]
           + jnp.dot(ea_ref[...], we_ref[...], preferred_element_type=jnp.float32))
    t1 = jax.nn.silu(pre)
    mij = jax.nn.silu(jnp.dot(t1, ew2_ref[...], preferred_element_type=jnp.float32)
                      + eb2_ref[...])
    mij_ref[...] = mij
    sv = jax.nn.silu(jnp.dot(mij, pw1_ref[...], preferred_element_type=jnp.float32)
                     + pb1_ref[...])
    wgt = jnp.tanh(jnp.sum(sv * pw2_ref[...], axis=1, keepdims=True))
    wv_ref[...] = reln * wgt


def _scatter_sc(mij_hbm, wv_hbm, dst_hbm, zmi_hbm, zpa_hbm, mi2_hbm, pa2_hbm,
                idx_d, mrows, wrows, smi, spa):
    E = dst_hbm.shape[0]
    N = zmi_hbm.shape[0]
    per = E // NW
    rows = N // NS
    c = lax.axis_index("c")
    s = lax.axis_index("s")
    wid = s * NC + c

    pltpu.sync_copy(zmi_hbm.at[pl.ds(s * rows, rows)], smi.at[pl.ds(s * rows, rows)])
    pltpu.sync_copy(zpa_hbm.at[pl.ds(s * rows, rows)], spa.at[pl.ds(s * rows, rows)])
    plsc.subcore_barrier()

    @pl.loop(0, per // CH)
    def _chunk(t):
        base = wid * per + t * CH
        pltpu.sync_copy(dst_hbm.at[pl.ds(base, CH)], idx_d)
        pltpu.sync_copy(mij_hbm.at[pl.ds(base, CH)], mrows)
        pltpu.sync_copy(wv_hbm.at[pl.ds(base, CH)], wrows)
        pltpu.sync_copy(mrows, smi.at[idx_d], add=True)
        pltpu.sync_copy(wrows, spa.at[idx_d], add=True)

    plsc.subcore_barrier()
    pltpu.sync_copy(smi.at[pl.ds(s * rows, rows)],
                    mi2_hbm.at[c, pl.ds(s * rows, rows)])
    pltpu.sync_copy(spa.at[pl.ds(s * rows, rows)],
                    pa2_hbm.at[c, pl.ds(s * rows, rows)])


def _final_body(nf_ref, p_ref, mi2_ref, pa2_ref, fw1a_ref, fw1b_ref, fb1_ref,
                fw2_ref, fb2_ref, feats_ref, pos_ref):
    nf = nf_ref[...]
    mi = mi2_ref[0] + mi2_ref[1]
    h = jax.nn.silu(jnp.dot(nf, fw1a_ref[...], preferred_element_type=jnp.float32)
                    + jnp.dot(mi, fw1b_ref[...], preferred_element_type=jnp.float32)
                    + fb1_ref[...])
    feats_ref[...] = (jnp.dot(h, fw2_ref[...], preferred_element_type=jnp.float32)
                      + fb2_ref[...] + nf)
    pos_ref[...] = p_ref[...] + pa2_ref[0] + pa2_ref[1]


def kernel(node_feats, positions, edge_attr, eW1, eb1, eW2, eb2,
           fW1, fb1, fW2, fb2, pW1, pb1, pW2, edge_index):
    N, D = node_feats.shape
    E = edge_index.shape[1]
    H = eW2.shape[0]
    f32 = jnp.float32

    src = edge_index[0]
    dst = edge_index[1]
    Ws = eW1[:D]
    Wd = eW1[D:2 * D]
    wr = eW1[2 * D:2 * D + 1]
    We = eW1[2 * D + 1:]
    P = jnp.pad(positions, ((0, 0), (0, 13)))

    BN = 2000
    A, B = pl.pallas_call(
        _prep_body,
        grid=(N // BN,),
        in_specs=[
            pl.BlockSpec((BN, D), lambda i: (i, 0)),
            pl.BlockSpec((D, H), lambda i: (0, 0)),
            pl.BlockSpec((D, H), lambda i: (0, 0)),
        ],
        out_specs=[
            pl.BlockSpec((BN, H), lambda i: (i, 0)),
            pl.BlockSpec((BN, H), lambda i: (i, 0)),
        ],
        out_shape=[
            jax.ShapeDtypeStruct((N, H), f32),
            jax.ShapeDtypeStruct((N, H), f32),
        ],
    )(node_feats, Ws, Wd)

    mesh = plsc.VectorSubcoreMesh(core_axis_name="c", subcore_axis_name="s")
    sc_params = pltpu.CompilerParams(use_tc_tiling_on_sc=False)
    gsum, rel = pl.kernel(
        _gather_sc,
        out_type=(jax.ShapeDtypeStruct((E, H), f32),
                  jax.ShapeDtypeStruct((E, 16), f32)),
        mesh=mesh,
        compiler_params=sc_params,
        scratch_types=[
            pltpu.VMEM((CH,), jnp.int32),
            pltpu.VMEM((CH,), jnp.int32),
            pltpu.VMEM((CH, H), f32),
            pltpu.VMEM((CH, H), f32),
            pltpu.VMEM((CH, 16), f32),
            pltpu.VMEM((CH, 16), f32),
            pltpu.VMEM((CH, 16), f32),
            pltpu.SemaphoreType.DMA,
            pltpu.SemaphoreType.DMA,
            pltpu.SemaphoreType.DMA,
            pltpu.SemaphoreType.DMA,
        ],
    )(A, B, P, src, dst)

    BE = 2000
    eb1r = eb1.reshape(1, H)
    eb2r = eb2.reshape(1, H)
    pb1r = pb1.reshape(1, H)
    pw2r = pW2.reshape(1, H)
    mij, wv = pl.pallas_call(
        _edge_body,
        grid=(E // BE,),
        in_specs=[
            pl.BlockSpec((BE, H), lambda i: (i, 0)),
            pl.BlockSpec((BE, 16), lambda i: (i, 0)),
            pl.BlockSpec((BE, 16), lambda i: (i, 0)),
            pl.BlockSpec((1, H), lambda i: (0, 0)),
            pl.BlockSpec((1, H), lambda i: (0, 0)),
            pl.BlockSpec((16, H), lambda i: (0, 0)),
            pl.BlockSpec((H, H), lambda i: (0, 0)),
            pl.BlockSpec((1, H), lambda i: (0, 0)),
            pl.BlockSpec((H, H), lambda i: (0, 0)),
            pl.BlockSpec((1, H), lambda i: (0, 0)),
            pl.BlockSpec((1, H), lambda i: (0, 0)),
        ],
        out_specs=[
            pl.BlockSpec((BE, H), lambda i: (i, 0)),
            pl.BlockSpec((BE, 16), lambda i: (i, 0)),
        ],
        out_shape=[
            jax.ShapeDtypeStruct((E, H), f32),
            jax.ShapeDtypeStruct((E, 16), f32),
        ],
    )(gsum, rel, edge_attr, eb1r, wr, We, eW2, eb2r, pW1, pb1r, pw2r)

    zmi = jnp.zeros((N, H), f32)
    zpa = jnp.zeros((N, 16), f32)
    mi2, pa2 = pl.kernel(
        _scatter_sc,
        out_type=(jax.ShapeDtypeStruct((NC, N, H), f32),
                  jax.ShapeDtypeStruct((NC, N, 16), f32)),
        mesh=mesh,
        compiler_params=sc_params,
        scratch_types=[
            pltpu.VMEM((CH,), jnp.int32),
            pltpu.VMEM((CH, H), f32),
            pltpu.VMEM((CH, 16), f32),
            pltpu.VMEM_SHARED((N, H), f32),
            pltpu.VMEM_SHARED((N, 16), f32),
        ],
    )(mij, wv, dst, zmi, zpa)

    fW1a = fW1[:D]
    fW1b = fW1[D:]
    fb1r = fb1.reshape(1, H)
    fb2r = fb2.reshape(1, D)
    feats, posp = pl.pallas_call(
        _final_body,
        grid=(N // BN,),
        in_specs=[
            pl.BlockSpec((BN, D), lambda i: (i, 0)),
            pl.BlockSpec((BN, 16), lambda i: (i, 0)),
            pl.BlockSpec((NC, BN, H), lambda i: (0, i, 0)),
            pl.BlockSpec((NC, BN, 16), lambda i: (0, i, 0)),
            pl.BlockSpec((D, H), lambda i: (0, 0)),
            pl.BlockSpec((H, H), lambda i: (0, 0)),
            pl.BlockSpec((1, H), lambda i: (0, 0)),
            pl.BlockSpec((H, D), lambda i: (0, 0)),
            pl.BlockSpec((1, D), lambda i: (0, 0)),
        ],
        out_specs=[
            pl.BlockSpec((BN, D), lambda i: (i, 0)),
            pl.BlockSpec((BN, 16), lambda i: (i, 0)),
        ],
        out_shape=[
            jax.ShapeDtypeStruct((N, D), f32),
            jax.ShapeDtypeStruct((N, 16), f32),
        ],
    )(node_feats, P, mi2, pa2, fW1a, fW1b, fb1r, fW2, fb2r)

    return (feats, posp[:, :3])


# speedup vs baseline: 4.1137x; 1.4224x over previous
import functools

import jax
import jax.numpy as jnp
from jax import lax
from jax.experimental import pallas as pl
from jax.experimental.pallas import tpu as pltpu
from jax.experimental.pallas import tpu_sc as plsc

NC = 2
NS = 16
NW = NC * NS
CH = 100
TW = 144


def _prep_body(nf_ref, ws_ref, wd_ref, a_ref, b_ref):
    nf = nf_ref[...]
    a_ref[...] = jnp.dot(nf, ws_ref[...], preferred_element_type=jnp.float32)
    b_ref[...] = jnp.dot(nf, wd_ref[...], preferred_element_type=jnp.float32)


def _gather_sc(ta_hbm, tb_hbm, src3, dst3, gsum_hbm, rel_hbm,
               idxs, idxd, ar0, ar1, br0, br1, os0, os1, rv0, rv1,
               g0, g1, w0, w1):
    E = gsum_hbm.shape[0]
    per = E // NW
    nch = per // CH
    wid = lax.axis_index("s") * NC + lax.axis_index("c")
    AR = (ar0, ar1)
    BR = (br0, br1)
    OS = (os0, os1)
    RV = (rv0, rv1)
    G = (g0, g1)
    W = (w0, w1)

    pltpu.sync_copy(src3.at[wid], idxs)
    pltpu.sync_copy(dst3.at[wid], idxd)

    def issue_gather(t, b):
        pltpu.async_copy(ta_hbm.at[idxs.at[t]], AR[b], G[b])
        pltpu.async_copy(tb_hbm.at[idxd.at[t]], BR[b], G[b])

    def wait_gather(t, b):
        pltpu.make_async_copy(ta_hbm.at[idxs.at[t]], AR[b], G[b]).wait()
        pltpu.make_async_copy(tb_hbm.at[idxd.at[t]], BR[b], G[b]).wait()

    def issue_wb(t, b):
        base = wid * per + t * CH
        pltpu.async_copy(OS[b], gsum_hbm.at[pl.ds(base, CH)], W[b])
        pltpu.async_copy(RV[b], rel_hbm.at[pl.ds(base, CH)], W[b])

    def wait_wb(t, b):
        base = wid * per + t * CH
        pltpu.make_async_copy(OS[b], gsum_hbm.at[pl.ds(base, CH)], W[b]).wait()
        pltpu.make_async_copy(RV[b], rel_hbm.at[pl.ds(base, CH)], W[b]).wait()

    issue_gather(0, 0)
    issue_gather(1, 1)

    @pl.loop(0, nch, step=2)
    def _steps(t):
        for b in range(2):
            tt = t + b
            wait_gather(tt, b)

            @pl.when(tt >= 2)
            def _():
                wait_wb(tt - 2, b)

            @pl.loop(0, CH)
            def _row(i):
                for j in range(8):
                    sl = pl.ds(j * 16, 16)
                    OS[b][i, sl] = AR[b][i, sl] + BR[b][i, sl]
                pw = pl.ds(128, 16)
                RV[b][i, :] = AR[b][i, pw] - BR[b][i, pw]

            @pl.when(tt + 2 < nch)
            def _():
                issue_gather(tt + 2, b)

            issue_wb(tt, b)

    wait_wb(nch - 2, 0)
    wait_wb(nch - 1, 1)


def _edge_body(gsum_ref, rel_ref, ea_ref, eb1_ref, wr_ref, we_ref, ew2_ref,
               eb2_ref, pw1_ref, pb1_ref, pw2_ref, mij_ref, wv_ref):
    rel = rel_ref[...]
    dist = jnp.sum(rel * rel, axis=1, keepdims=True)
    r = jnp.sqrt(dist)
    reln = rel / (r + 0.1)
    pre = (gsum_ref[...] + eb1_ref[...] + dist * wr_ref[...]
           + jnp.dot(ea_ref[...], we_ref[...], preferred_element_type=jnp.float32))
    t1 = jax.nn.silu(pre)
    mij = jax.nn.silu(jnp.dot(t1, ew2_ref[...], preferred_element_type=jnp.float32)
                      + eb2_ref[...])
    mij_ref[...] = mij
    sv = jax.nn.silu(jnp.dot(mij, pw1_ref[...], preferred_element_type=jnp.float32)
                     + pb1_ref[...])
    wgt = jnp.tanh(jnp.sum(sv * pw2_ref[...], axis=1, keepdims=True))
    wv_ref[...] = reln * wgt


def _scatter_sc(mij_hbm, wv_hbm, dst3, mi2_hbm, pa2_hbm,
                idxd, mr0, mr1, wr0, wr1, smi, spa,
                l0, l1, s0, s1):
    E = mij_hbm.shape[0]
    N = mi2_hbm.shape[1]
    per = E // NW
    nch = per // CH
    rows = N // NS
    c = lax.axis_index("c")
    s = lax.axis_index("s")
    wid = s * NC + c
    MR = (mr0, mr1)
    WR = (wr0, wr1)
    L = (l0, l1)
    S = (s0, s1)
    pltpu.sync_copy(dst3.at[wid], idxd)

    z16 = jnp.zeros((16,), jnp.float32)

    @pl.loop(0, CH)
    def _z(i):
        for j in range(8):
            mr0[i, pl.ds(j * 16, 16)] = z16
        wr0[i, :] = z16

    for k in range(rows // CH):
        pltpu.sync_copy(mr0, smi.at[pl.ds(s * rows + k * CH, CH)])
        pltpu.sync_copy(wr0, spa.at[pl.ds(s * rows + k * CH, CH)])
    tail = rows % CH
    if tail:
        pltpu.sync_copy(mr0.at[pl.ds(0, tail)],
                        smi.at[pl.ds(s * rows + rows - tail, tail)])
        pltpu.sync_copy(wr0.at[pl.ds(0, tail)],
                        spa.at[pl.ds(s * rows + rows - tail, tail)])
    plsc.subcore_barrier()

    def issue_load(t, b):
        base = wid * per + t * CH
        pltpu.async_copy(mij_hbm.at[pl.ds(base, CH)], MR[b], L[b])
        pltpu.async_copy(wv_hbm.at[pl.ds(base, CH)], WR[b], L[b])

    def wait_load(t, b):
        base = wid * per + t * CH
        pltpu.make_async_copy(mij_hbm.at[pl.ds(base, CH)], MR[b], L[b]).wait()
        pltpu.make_async_copy(wv_hbm.at[pl.ds(base, CH)], WR[b], L[b]).wait()

    issue_load(0, 0)
    issue_load(1, 1)

    @pl.loop(0, nch, step=2)
    def _steps(t):
        for b in range(2):
            tt = t + b
            wait_load(tt, b)
            cm = pltpu.async_copy(MR[b], smi.at[idxd.at[tt]], S[b], add=True)
            cw = pltpu.async_copy(WR[b], spa.at[idxd.at[tt]], S[b], add=True)
            cm.wait()
            cw.wait()

            @pl.when(tt + 2 < nch)
            def _():
                issue_load(tt + 2, b)

    plsc.subcore_barrier()
    pltpu.sync_copy(smi.at[pl.ds(s * rows, rows)],
                    mi2_hbm.at[c, pl.ds(s * rows, rows)])
    pltpu.sync_copy(spa.at[pl.ds(s * rows, rows)],
                    pa2_hbm.at[c, pl.ds(s * rows, rows)])


def _final_body(nf_ref, p_ref, mi2_ref, pa2_ref, fw1a_ref, fw1b_ref, fb1_ref,
                fw2_ref, fb2_ref, feats_ref, pos_ref):
    nf = nf_ref[...]
    mi = mi2_ref[0] + mi2_ref[1]
    h = jax.nn.silu(jnp.dot(nf, fw1a_ref[...], preferred_element_type=jnp.float32)
                    + jnp.dot(mi, fw1b_ref[...], preferred_element_type=jnp.float32)
                    + fb1_ref[...])
    feats_ref[...] = (jnp.dot(h, fw2_ref[...], preferred_element_type=jnp.float32)
                      + fb2_ref[...] + nf)
    pos_ref[...] = p_ref[...] + pa2_ref[0] + pa2_ref[1]


def kernel(node_feats, positions, edge_attr, eW1, eb1, eW2, eb2,
           fW1, fb1, fW2, fb2, pW1, pb1, pW2, edge_index):
    N, D = node_feats.shape
    E = edge_index.shape[1]
    H = eW2.shape[0]
    f32 = jnp.float32
    per = E // NW
    nch = per // CH

    src3 = edge_index[0].reshape(NW, nch, CH)
    dst3 = edge_index[1].reshape(NW, nch, CH)
    Ws = eW1[:D]
    Wd = eW1[D:2 * D]
    wr = eW1[2 * D:2 * D + 1]
    We = eW1[2 * D + 1:]
    P = jnp.pad(positions, ((0, 0), (0, 13)))

    BN = 2000
    A, B = pl.pallas_call(
        _prep_body,
        grid=(N // BN,),
        in_specs=[
            pl.BlockSpec((BN, D), lambda i: (i, 0)),
            pl.BlockSpec((D, H), lambda i: (0, 0)),
            pl.BlockSpec((D, H), lambda i: (0, 0)),
        ],
        out_specs=[
            pl.BlockSpec((BN, H), lambda i: (i, 0)),
            pl.BlockSpec((BN, H), lambda i: (i, 0)),
        ],
        out_shape=[
            jax.ShapeDtypeStruct((N, H), f32),
            jax.ShapeDtypeStruct((N, H), f32),
        ],
    )(node_feats, Ws, Wd)

    TA = jnp.concatenate([A, P], axis=1)
    TB = jnp.concatenate([B, P], axis=1)

    mesh = plsc.VectorSubcoreMesh(core_axis_name="c", subcore_axis_name="s")
    sc_params = pltpu.CompilerParams(use_tc_tiling_on_sc=False)
    gather_scratch = [
        pltpu.VMEM((nch, CH), jnp.int32),
        pltpu.VMEM((nch, CH), jnp.int32),
        pltpu.VMEM((CH, TW), f32),
        pltpu.VMEM((CH, TW), f32),
        pltpu.VMEM((CH, TW), f32),
        pltpu.VMEM((CH, TW), f32),
        pltpu.VMEM((CH, H), f32),
        pltpu.VMEM((CH, H), f32),
        pltpu.VMEM((CH, 16), f32),
        pltpu.VMEM((CH, 16), f32),
        pltpu.SemaphoreType.DMA,
        pltpu.SemaphoreType.DMA,
        pltpu.SemaphoreType.DMA,
        pltpu.SemaphoreType.DMA,
    ]
    gsum, rel = pl.kernel(
        _gather_sc,
        out_type=(jax.ShapeDtypeStruct((E, H), f32),
                  jax.ShapeDtypeStruct((E, 16), f32)),
        mesh=mesh,
        compiler_params=sc_params,
        scratch_types=gather_scratch,
    )(TA, TB, src3, dst3)

    BE = 2000
    eb1r = eb1.reshape(1, H)
    eb2r = eb2.reshape(1, H)
    pb1r = pb1.reshape(1, H)
    pw2r = pW2.reshape(1, H)
    mij, wv = pl.pallas_call(
        _edge_body,
        grid=(E // BE,),
        in_specs=[
            pl.BlockSpec((BE, H), lambda i: (i, 0)),
            pl.BlockSpec((BE, 16), lambda i: (i, 0)),
            pl.BlockSpec((BE, 16), lambda i: (i, 0)),
            pl.BlockSpec((1, H), lambda i: (0, 0)),
            pl.BlockSpec((1, H), lambda i: (0, 0)),
            pl.BlockSpec((16, H), lambda i: (0, 0)),
            pl.BlockSpec((H, H), lambda i: (0, 0)),
            pl.BlockSpec((1, H), lambda i: (0, 0)),
            pl.BlockSpec((H, H), lambda i: (0, 0)),
            pl.BlockSpec((1, H), lambda i: (0, 0)),
            pl.BlockSpec((1, H), lambda i: (0, 0)),
        ],
        out_specs=[
            pl.BlockSpec((BE, H), lambda i: (i, 0)),
            pl.BlockSpec((BE, 16), lambda i: (i, 0)),
        ],
        out_shape=[
            jax.ShapeDtypeStruct((E, H), f32),
            jax.ShapeDtypeStruct((E, 16), f32),
        ],
    )(gsum, rel, edge_attr, eb1r, wr, We, eW2, eb2r, pW1, pb1r, pw2r)

    scatter_scratch = [
        pltpu.VMEM((nch, CH), jnp.int32),
        pltpu.VMEM((CH, H), f32),
        pltpu.VMEM((CH, H), f32),
        pltpu.VMEM((CH, 16), f32),
        pltpu.VMEM((CH, 16), f32),
        pltpu.VMEM_SHARED((N, H), f32),
        pltpu.VMEM_SHARED((N, 16), f32),
        pltpu.SemaphoreType.DMA,
        pltpu.SemaphoreType.DMA,
        pltpu.SemaphoreType.DMA,
        pltpu.SemaphoreType.DMA,
    ]
    mi2, pa2 = pl.kernel(
        _scatter_sc,
        out_type=(jax.ShapeDtypeStruct((NC, N, H), f32),
                  jax.ShapeDtypeStruct((NC, N, 16), f32)),
        mesh=mesh,
        compiler_params=sc_params,
        scratch_types=scatter_scratch,
    )(mij, wv, dst3)

    fW1a = fW1[:D]
    fW1b = fW1[D:]
    fb1r = fb1.reshape(1, H)
    fb2r = fb2.reshape(1, D)
    feats, posp = pl.pallas_call(
        _final_body,
        grid=(N // BN,),
        in_specs=[
            pl.BlockSpec((BN, D), lambda i: (i, 0)),
            pl.BlockSpec((BN, 16), lambda i: (i, 0)),
            pl.BlockSpec((NC, BN, H), lambda i: (0, i, 0)),
            pl.BlockSpec((NC, BN, 16), lambda i: (0, i, 0)),
            pl.BlockSpec((D, H), lambda i: (0, 0)),
            pl.BlockSpec((H, H), lambda i: (0, 0)),
            pl.BlockSpec((1, H), lambda i: (0, 0)),
            pl.BlockSpec((H, D), lambda i: (0, 0)),
            pl.BlockSpec((1, D), lambda i: (0, 0)),
        ],
        out_specs=[
            pl.BlockSpec((BN, D), lambda i: (i, 0)),
            pl.BlockSpec((BN, 16), lambda i: (i, 0)),
        ],
        out_shape=[
            jax.ShapeDtypeStruct((N, D), f32),
            jax.ShapeDtypeStruct((N, 16), f32),
        ],
    )(node_feats, P, mi2, pa2, fW1a, fW1b, fb1r, fW2, fb2r)

    return (feats, posp[:, :3])


# speedup vs baseline: 4.2851x; 1.0417x over previous
import functools

import jax
import jax.numpy as jnp
from jax import lax
from jax.experimental import pallas as pl
from jax.experimental.pallas import tpu as pltpu
from jax.experimental.pallas import tpu_sc as plsc

NC = 2
NS = 16
NW = NC * NS
CH = 100
TW = 144


def _prep_body(nf_ref, ws_ref, wd_ref, a_ref, b_ref):
    nf = nf_ref[...]
    a_ref[...] = jnp.dot(nf, ws_ref[...], preferred_element_type=jnp.float32)
    b_ref[...] = jnp.dot(nf, wd_ref[...], preferred_element_type=jnp.float32)


def _gather_sc(ta_hbm, tb_hbm, src3, dst3, gsum_hbm, rel_hbm,
               idxs, idxd, ar0, ar1, br0, br1, os0, os1, rv0, rv1,
               g0, g1, w0, w1):
    E = gsum_hbm.shape[0]
    per = E // NW
    nch = per // CH
    wid = lax.axis_index("s") * NC + lax.axis_index("c")
    AR = (ar0, ar1)
    BR = (br0, br1)
    OS = (os0, os1)
    RV = (rv0, rv1)
    G = (g0, g1)
    W = (w0, w1)

    pltpu.sync_copy(src3.at[wid], idxs)
    pltpu.sync_copy(dst3.at[wid], idxd)

    def issue_gather(t, b):
        pltpu.async_copy(ta_hbm.at[idxs.at[t]], AR[b], G[b])
        pltpu.async_copy(tb_hbm.at[idxd.at[t]], BR[b], G[b])

    def wait_gather(t, b):
        pltpu.make_async_copy(ta_hbm.at[idxs.at[t]], AR[b], G[b]).wait()
        pltpu.make_async_copy(tb_hbm.at[idxd.at[t]], BR[b], G[b]).wait()

    def issue_wb(t, b):
        base = wid * per + t * CH
        pltpu.async_copy(OS[b], gsum_hbm.at[pl.ds(base, CH)], W[b])
        pltpu.async_copy(RV[b], rel_hbm.at[pl.ds(base, CH)], W[b])

    def wait_wb(t, b):
        base = wid * per + t * CH
        pltpu.make_async_copy(OS[b], gsum_hbm.at[pl.ds(base, CH)], W[b]).wait()
        pltpu.make_async_copy(RV[b], rel_hbm.at[pl.ds(base, CH)], W[b]).wait()

    issue_gather(0, 0)
    issue_gather(1, 1)

    @pl.loop(0, nch, step=2)
    def _steps(t):
        for b in range(2):
            tt = t + b
            wait_gather(tt, b)

            @pl.when(tt >= 2)
            def _():
                wait_wb(tt - 2, b)

            @pl.loop(0, CH)
            def _row(i):
                for j in range(8):
                    sl = pl.ds(j * 16, 16)
                    OS[b][i, sl] = AR[b][i, sl] + BR[b][i, sl]
                pw = pl.ds(128, 16)
                RV[b][i, :] = AR[b][i, pw] - BR[b][i, pw]

            @pl.when(tt + 2 < nch)
            def _():
                issue_gather(tt + 2, b)

            issue_wb(tt, b)

    wait_wb(nch - 2, 0)
    wait_wb(nch - 1, 1)


def _edge_body(gsum_ref, rel_ref, eaT_ref, eb1_ref, wr_ref, we_ref, ew2_ref,
               eb2_ref, pw1_ref, pb1_ref, pw2_ref, mij_ref, wv_ref):
    rel = rel_ref[...]
    dist = jnp.sum(rel * rel, axis=1, keepdims=True)
    r = jnp.sqrt(dist)
    reln = rel / (r + 0.1)
    ea_term = lax.dot_general(eaT_ref[...], we_ref[...],
                              (((0,), (0,)), ((), ())),
                              preferred_element_type=jnp.float32)
    pre = gsum_ref[...] + eb1_ref[...] + dist * wr_ref[...] + ea_term
    t1 = jax.nn.silu(pre)
    mij = jax.nn.silu(jnp.dot(t1, ew2_ref[...], preferred_element_type=jnp.float32)
                      + eb2_ref[...])
    mij_ref[...] = mij
    sv = jax.nn.silu(jnp.dot(mij, pw1_ref[...], preferred_element_type=jnp.float32)
                     + pb1_ref[...])
    wgt = jnp.tanh(jnp.sum(sv * pw2_ref[...], axis=1, keepdims=True))
    wv_ref[...] = reln * wgt


def _scatter_sc(mij_hbm, wv_hbm, dst3, mi2_hbm, pa2_hbm,
                idxd, mr0, mr1, wr0, wr1, smi, spa,
                l0, l1, s0, s1):
    E = mij_hbm.shape[0]
    N = mi2_hbm.shape[1]
    per = E // NW
    nch = per // CH
    rows = N // NS
    c = lax.axis_index("c")
    s = lax.axis_index("s")
    wid = s * NC + c
    MR = (mr0, mr1)
    WR = (wr0, wr1)
    L = (l0, l1)
    S = (s0, s1)
    pltpu.sync_copy(dst3.at[wid], idxd)

    z16 = jnp.zeros((16,), jnp.float32)

    @pl.loop(0, CH)
    def _z(i):
        for j in range(8):
            mr0[i, pl.ds(j * 16, 16)] = z16
        wr0[i, :] = z16

    for k in range(rows // CH):
        pltpu.sync_copy(mr0, smi.at[pl.ds(s * rows + k * CH, CH)])
        pltpu.sync_copy(wr0, spa.at[pl.ds(s * rows + k * CH, CH)])
    tail = rows % CH
    if tail:
        pltpu.sync_copy(mr0.at[pl.ds(0, tail)],
                        smi.at[pl.ds(s * rows + rows - tail, tail)])
        pltpu.sync_copy(wr0.at[pl.ds(0, tail)],
                        spa.at[pl.ds(s * rows + rows - tail, tail)])
    plsc.subcore_barrier()

    def issue_load(t, b):
        base = wid * per + t * CH
        pltpu.async_copy(mij_hbm.at[pl.ds(base, CH)], MR[b], L[b])
        pltpu.async_copy(wv_hbm.at[pl.ds(base, CH)], WR[b], L[b])

    def wait_load(t, b):
        base = wid * per + t * CH
        pltpu.make_async_copy(mij_hbm.at[pl.ds(base, CH)], MR[b], L[b]).wait()
        pltpu.make_async_copy(wv_hbm.at[pl.ds(base, CH)], WR[b], L[b]).wait()

    issue_load(0, 0)
    issue_load(1, 1)

    @pl.loop(0, nch, step=2)
    def _steps(t):
        for b in range(2):
            tt = t + b
            wait_load(tt, b)
            cm = pltpu.async_copy(MR[b], smi.at[idxd.at[tt]], S[b], add=True)
            cw = pltpu.async_copy(WR[b], spa.at[idxd.at[tt]], S[b], add=True)
            cm.wait()
            cw.wait()

            @pl.when(tt + 2 < nch)
            def _():
                issue_load(tt + 2, b)

    plsc.subcore_barrier()
    pltpu.sync_copy(smi.at[pl.ds(s * rows, rows)],
                    mi2_hbm.at[c, pl.ds(s * rows, rows)])
    pltpu.sync_copy(spa.at[pl.ds(s * rows, rows)],
                    pa2_hbm.at[c, pl.ds(s * rows, rows)])


def _final_body(nf_ref, p_ref, mi2_ref, pa2_ref, fw1a_ref, fw1b_ref, fb1_ref,
                fw2_ref, fb2_ref, feats_ref, pos_ref):
    nf = nf_ref[...]
    mi = mi2_ref[0] + mi2_ref[1]
    h = jax.nn.silu(jnp.dot(nf, fw1a_ref[...], preferred_element_type=jnp.float32)
                    + jnp.dot(mi, fw1b_ref[...], preferred_element_type=jnp.float32)
                    + fb1_ref[...])
    feats_ref[...] = (jnp.dot(h, fw2_ref[...], preferred_element_type=jnp.float32)
                      + fb2_ref[...] + nf)
    pos_ref[...] = p_ref[...] + pa2_ref[0] + pa2_ref[1]


def kernel(node_feats, positions, edge_attr, eW1, eb1, eW2, eb2,
           fW1, fb1, fW2, fb2, pW1, pb1, pW2, edge_index):
    N, D = node_feats.shape
    E = edge_index.shape[1]
    H = eW2.shape[0]
    f32 = jnp.float32
    per = E // NW
    nch = per // CH

    src3 = edge_index[0].reshape(NW, nch, CH)
    dst3 = edge_index[1].reshape(NW, nch, CH)
    Ws = eW1[:D]
    Wd = eW1[D:2 * D]
    wr = eW1[2 * D:2 * D + 1]
    We = eW1[2 * D + 1:]
    eaT = edge_attr.T
    P = jnp.pad(positions, ((0, 0), (0, 13)))

    BN = 2000
    A, B = pl.pallas_call(
        _prep_body,
        grid=(N // BN,),
        in_specs=[
            pl.BlockSpec((BN, D), lambda i: (i, 0)),
            pl.BlockSpec((D, H), lambda i: (0, 0)),
            pl.BlockSpec((D, H), lambda i: (0, 0)),
        ],
        out_specs=[
            pl.BlockSpec((BN, H), lambda i: (i, 0)),
            pl.BlockSpec((BN, H), lambda i: (i, 0)),
        ],
        out_shape=[
            jax.ShapeDtypeStruct((N, H), f32),
            jax.ShapeDtypeStruct((N, H), f32),
        ],
    )(node_feats, Ws, Wd)

    TA = jnp.concatenate([A, P], axis=1)
    TB = jnp.concatenate([B, P], axis=1)

    mesh = plsc.VectorSubcoreMesh(core_axis_name="c", subcore_axis_name="s")
    sc_params = pltpu.CompilerParams(use_tc_tiling_on_sc=False)
    gather_scratch = [
        pltpu.VMEM((nch, CH), jnp.int32),
        pltpu.VMEM((nch, CH), jnp.int32),
        pltpu.VMEM((CH, TW), f32),
        pltpu.VMEM((CH, TW), f32),
        pltpu.VMEM((CH, TW), f32),
        pltpu.VMEM((CH, TW), f32),
        pltpu.VMEM((CH, H), f32),
        pltpu.VMEM((CH, H), f32),
        pltpu.VMEM((CH, 16), f32),
        pltpu.VMEM((CH, 16), f32),
        pltpu.SemaphoreType.DMA,
        pltpu.SemaphoreType.DMA,
        pltpu.SemaphoreType.DMA,
        pltpu.SemaphoreType.DMA,
    ]
    gsum, rel = pl.kernel(
        _gather_sc,
        out_type=(jax.ShapeDtypeStruct((E, H), f32),
                  jax.ShapeDtypeStruct((E, 16), f32)),
        mesh=mesh,
        compiler_params=sc_params,
        scratch_types=gather_scratch,
    )(TA, TB, src3, dst3)

    BE = 3200
    eb1r = eb1.reshape(1, H)
    eb2r = eb2.reshape(1, H)
    pb1r = pb1.reshape(1, H)
    pw2r = pW2.reshape(1, H)
    mij, wv = pl.pallas_call(
        _edge_body,
        grid=(E // BE,),
        in_specs=[
            pl.BlockSpec((BE, H), lambda i: (i, 0)),
            pl.BlockSpec((BE, 16), lambda i: (i, 0)),
            pl.BlockSpec((16, BE), lambda i: (0, i)),
            pl.BlockSpec((1, H), lambda i: (0, 0)),
            pl.BlockSpec((1, H), lambda i: (0, 0)),
            pl.BlockSpec((16, H), lambda i: (0, 0)),
            pl.BlockSpec((H, H), lambda i: (0, 0)),
            pl.BlockSpec((1, H), lambda i: (0, 0)),
            pl.BlockSpec((H, H), lambda i: (0, 0)),
            pl.BlockSpec((1, H), lambda i: (0, 0)),
            pl.BlockSpec((1, H), lambda i: (0, 0)),
        ],
        out_specs=[
            pl.BlockSpec((BE, H), lambda i: (i, 0)),
            pl.BlockSpec((BE, 16), lambda i: (i, 0)),
        ],
        out_shape=[
            jax.ShapeDtypeStruct((E, H), f32),
            jax.ShapeDtypeStruct((E, 16), f32),
        ],
    )(gsum, rel, eaT, eb1r, wr, We, eW2, eb2r, pW1, pb1r, pw2r)

    scatter_scratch = [
        pltpu.VMEM((nch, CH), jnp.int32),
        pltpu.VMEM((CH, H), f32),
        pltpu.VMEM((CH, H), f32),
        pltpu.VMEM((CH, 16), f32),
        pltpu.VMEM((CH, 16), f32),
        pltpu.VMEM_SHARED((N, H), f32),
        pltpu.VMEM_SHARED((N, 16), f32),
        pltpu.SemaphoreType.DMA,
        pltpu.SemaphoreType.DMA,
        pltpu.SemaphoreType.DMA,
        pltpu.SemaphoreType.DMA,
    ]
    mi2, pa2 = pl.kernel(
        _scatter_sc,
        out_type=(jax.ShapeDtypeStruct((NC, N, H), f32),
                  jax.ShapeDtypeStruct((NC, N, 16), f32)),
        mesh=mesh,
        compiler_params=sc_params,
        scratch_types=scatter_scratch,
    )(mij, wv, dst3)

    fW1a = fW1[:D]
    fW1b = fW1[D:]
    fb1r = fb1.reshape(1, H)
    fb2r = fb2.reshape(1, D)
    feats, posp = pl.pallas_call(
        _final_body,
        grid=(N // BN,),
        in_specs=[
            pl.BlockSpec((BN, D), lambda i: (i, 0)),
            pl.BlockSpec((BN, 16), lambda i: (i, 0)),
            pl.BlockSpec((NC, BN, H), lambda i: (0, i, 0)),
            pl.BlockSpec((NC, BN, 16), lambda i: (0, i, 0)),
            pl.BlockSpec((D, H), lambda i: (0, 0)),
            pl.BlockSpec((H, H), lambda i: (0, 0)),
            pl.BlockSpec((1, H), lambda i: (0, 0)),
            pl.BlockSpec((H, D), lambda i: (0, 0)),
            pl.BlockSpec((1, D), lambda i: (0, 0)),
        ],
        out_specs=[
            pl.BlockSpec((BN, D), lambda i: (i, 0)),
            pl.BlockSpec((BN, 16), lambda i: (i, 0)),
        ],
        out_shape=[
            jax.ShapeDtypeStruct((N, D), f32),
            jax.ShapeDtypeStruct((N, 16), f32),
        ],
    )(node_feats, P, mi2, pa2, fW1a, fW1b, fb1r, fW2, fb2r)

    return (feats, posp[:, :3])


# speedup vs baseline: 5.0189x; 1.1713x over previous
import functools

import jax
import jax.numpy as jnp
from jax import lax
from jax.experimental import pallas as pl
from jax.experimental.pallas import tpu as pltpu
from jax.experimental.pallas import tpu_sc as plsc

NC = 2
NS = 16
NW = NC * NS
CH = 100
TW = 144


def _prep_body(nf_ref, ws_ref, wd_ref, a_ref, b_ref):
    nf = nf_ref[...]
    a_ref[...] = jnp.dot(nf, ws_ref[...], preferred_element_type=jnp.float32)
    b_ref[...] = jnp.dot(nf, wd_ref[...], preferred_element_type=jnp.float32)


def _gather_sc(ta_hbm, tb_hbm, src3, dst3, gsum_hbm, rel_hbm,
               idxs, idxd, ar0, ar1, br0, br1, os0, os1, rv0, rv1,
               g0, g1, w0, w1):
    E = gsum_hbm.shape[0]
    per = E // NW
    nch = per // CH
    wid = lax.axis_index("s") * NC + lax.axis_index("c")
    AR = (ar0, ar1)
    BR = (br0, br1)
    OS = (os0, os1)
    RV = (rv0, rv1)
    G = (g0, g1)
    W = (w0, w1)

    pltpu.sync_copy(src3.at[wid], idxs)
    pltpu.sync_copy(dst3.at[wid], idxd)

    def issue_gather(t, b):
        pltpu.async_copy(ta_hbm.at[idxs.at[t]], AR[b], G[b])
        pltpu.async_copy(tb_hbm.at[idxd.at[t]], BR[b], G[b])

    def wait_gather(t, b):
        pltpu.make_async_copy(ta_hbm.at[idxs.at[t]], AR[b], G[b]).wait()
        pltpu.make_async_copy(tb_hbm.at[idxd.at[t]], BR[b], G[b]).wait()

    def issue_wb(t, b):
        base = wid * per + t * CH
        pltpu.async_copy(OS[b], gsum_hbm.at[pl.ds(base, CH)], W[b])
        pltpu.async_copy(RV[b], rel_hbm.at[pl.ds(base, CH)], W[b])

    def wait_wb(t, b):
        base = wid * per + t * CH
        pltpu.make_async_copy(OS[b], gsum_hbm.at[pl.ds(base, CH)], W[b]).wait()
        pltpu.make_async_copy(RV[b], rel_hbm.at[pl.ds(base, CH)], W[b]).wait()

    issue_gather(0, 0)
    issue_gather(1, 1)

    @pl.loop(0, nch, step=2)
    def _steps(t):
        for b in range(2):
            tt = t + b
            wait_gather(tt, b)

            @pl.when(tt >= 2)
            def _():
                wait_wb(tt - 2, b)

            @pl.loop(0, CH)
            def _row(i):
                for j in range(8):
                    sl = pl.ds(j * 16, 16)
                    OS[b][i, sl] = AR[b][i, sl] + BR[b][i, sl]
                pw = pl.ds(128, 16)
                RV[b][i, :] = AR[b][i, pw] - BR[b][i, pw]

            @pl.when(tt + 2 < nch)
            def _():
                issue_gather(tt + 2, b)

            issue_wb(tt, b)

    wait_wb(nch - 2, 0)
    wait_wb(nch - 1, 1)


def _edge_body(gsum_ref, rel_ref, eaT_ref, eb1_ref, wr_ref, we_ref, ew2_ref,
               eb2_ref, pw1_ref, pb1_ref, pw2_ref, mij_ref, wv_ref):
    rel = rel_ref[...]
    dist = jnp.sum(rel * rel, axis=1, keepdims=True)
    r = jnp.sqrt(dist)
    reln = rel / (r + 0.1)
    ea_term = lax.dot_general(eaT_ref[...], we_ref[...],
                              (((0,), (0,)), ((), ())),
                              preferred_element_type=jnp.float32)
    pre = gsum_ref[...] + eb1_ref[...] + dist * wr_ref[...] + ea_term
    t1 = jax.nn.silu(pre)
    mij = jax.nn.silu(jnp.dot(t1, ew2_ref[...], preferred_element_type=jnp.float32)
                      + eb2_ref[...])
    mij_ref[...] = mij
    sv = jax.nn.silu(jnp.dot(mij, pw1_ref[...], preferred_element_type=jnp.float32)
                     + pb1_ref[...])
    wgt = jnp.tanh(jnp.sum(sv * pw2_ref[...], axis=1, keepdims=True))
    wv_ref[...] = reln * wgt


def _scatter_sc(mijA, mijB, wvA, wvB, dstA3, dstB3, mi2_hbm, pa2_hbm,
                idxdA, idxdB, mr0, mr1, wr0, wr1, smi, spa,
                l0, l1, s0, s1):
    E = mijA.shape[0]
    N = mi2_hbm.shape[1]
    per = E // NW
    nch = per // CH
    rows = N // NS
    c = lax.axis_index("c")
    s = lax.axis_index("s")
    wid = s * NC + c
    MR = (mr0, mr1)
    WR = (wr0, wr1)
    L = (l0, l1)
    S = (s0, s1)
    pltpu.sync_copy(dstA3.at[wid], idxdA)
    pltpu.sync_copy(dstB3.at[wid], idxdB)

    z16 = jnp.zeros((16,), jnp.float32)

    @pl.loop(0, CH)
    def _z(i):
        for j in range(8):
            mr0[i, pl.ds(j * 16, 16)] = z16
        wr0[i, :] = z16

    for k in range(rows // CH):
        pltpu.sync_copy(mr0, smi.at[pl.ds(s * rows + k * CH, CH)])
        pltpu.sync_copy(wr0, spa.at[pl.ds(s * rows + k * CH, CH)])
    tail = rows % CH
    if tail:
        pltpu.sync_copy(mr0.at[pl.ds(0, tail)],
                        smi.at[pl.ds(s * rows + rows - tail, tail)])
        pltpu.sync_copy(wr0.at[pl.ds(0, tail)],
                        spa.at[pl.ds(s * rows + rows - tail, tail)])
    plsc.subcore_barrier()

    for mij_hbm_h, wv_hbm_h, idxd_h in ((mijA, wvA, idxdA), (mijB, wvB, idxdB)):
        def issue_load(t, b, m=mij_hbm_h, w=wv_hbm_h):
            base = wid * per + t * CH
            pltpu.async_copy(m.at[pl.ds(base, CH)], MR[b], L[b])
            pltpu.async_copy(w.at[pl.ds(base, CH)], WR[b], L[b])

        def wait_load(t, b, m=mij_hbm_h, w=wv_hbm_h):
            base = wid * per + t * CH
            pltpu.make_async_copy(m.at[pl.ds(base, CH)], MR[b], L[b]).wait()
            pltpu.make_async_copy(w.at[pl.ds(base, CH)], WR[b], L[b]).wait()

        issue_load(0, 0)
        issue_load(1, 1)

        @pl.loop(0, nch, step=2)
        def _steps(t, idxd_h=idxd_h, issue_load=issue_load, wait_load=wait_load):
            for b in range(2):
                tt = t + b
                wait_load(tt, b)
                cm = pltpu.async_copy(MR[b], smi.at[idxd_h.at[tt]], S[b], add=True)
                cw = pltpu.async_copy(WR[b], spa.at[idxd_h.at[tt]], S[b], add=True)
                cm.wait()
                cw.wait()

                @pl.when(tt + 2 < nch)
                def _():
                    issue_load(tt + 2, b)

    plsc.subcore_barrier()
    pltpu.sync_copy(smi.at[pl.ds(s * rows, rows)],
                    mi2_hbm.at[c, pl.ds(s * rows, rows)])
    pltpu.sync_copy(spa.at[pl.ds(s * rows, rows)],
                    pa2_hbm.at[c, pl.ds(s * rows, rows)])


def _final_body(nf_ref, p_ref, mi2_ref, pa2_ref, fw1a_ref, fw1b_ref, fb1_ref,
                fw2_ref, fb2_ref, feats_ref, pos_ref):
    nf = nf_ref[...]
    mi = mi2_ref[0] + mi2_ref[1]
    h = jax.nn.silu(jnp.dot(nf, fw1a_ref[...], preferred_element_type=jnp.float32)
                    + jnp.dot(mi, fw1b_ref[...], preferred_element_type=jnp.float32)
                    + fb1_ref[...])
    feats_ref[...] = (jnp.dot(h, fw2_ref[...], preferred_element_type=jnp.float32)
                      + fb2_ref[...] + nf)
    pos_ref[...] = p_ref[...] + pa2_ref[0] + pa2_ref[1]


def kernel(node_feats, positions, edge_attr, eW1, eb1, eW2, eb2,
           fW1, fb1, fW2, fb2, pW1, pb1, pW2, edge_index):
    N, D = node_feats.shape
    E = edge_index.shape[1]
    H = eW2.shape[0]
    f32 = jnp.float32
    E2 = E // 2
    per = E2 // NW
    nch = per // CH

    srcH = [edge_index[0, :E2].reshape(NW, nch, CH),
            edge_index[0, E2:].reshape(NW, nch, CH)]
    dstH = [edge_index[1, :E2].reshape(NW, nch, CH),
            edge_index[1, E2:].reshape(NW, nch, CH)]
    eaTH = [edge_attr[:E2].T, edge_attr[E2:].T]
    Ws = eW1[:D]
    Wd = eW1[D:2 * D]
    wr = eW1[2 * D:2 * D + 1]
    We = eW1[2 * D + 1:]
    eaT = edge_attr.T
    P = jnp.pad(positions, ((0, 0), (0, 13)))

    BN = 2000
    A, B = pl.pallas_call(
        _prep_body,
        grid=(N // BN,),
        in_specs=[
            pl.BlockSpec((BN, D), lambda i: (i, 0)),
            pl.BlockSpec((D, H), lambda i: (0, 0)),
            pl.BlockSpec((D, H), lambda i: (0, 0)),
        ],
        out_specs=[
            pl.BlockSpec((BN, H), lambda i: (i, 0)),
            pl.BlockSpec((BN, H), lambda i: (i, 0)),
        ],
        out_shape=[
            jax.ShapeDtypeStruct((N, H), f32),
            jax.ShapeDtypeStruct((N, H), f32),
        ],
    )(node_feats, Ws, Wd)

    TA = jnp.concatenate([A, P], axis=1)
    TB = jnp.concatenate([B, P], axis=1)

    mesh = plsc.VectorSubcoreMesh(core_axis_name="c", subcore_axis_name="s")
    sc_params = pltpu.CompilerParams(use_tc_tiling_on_sc=False)
    gather_scratch = [
        pltpu.VMEM((nch, CH), jnp.int32),
        pltpu.VMEM((nch, CH), jnp.int32),
        pltpu.VMEM((CH, TW), f32),
        pltpu.VMEM((CH, TW), f32),
        pltpu.VMEM((CH, TW), f32),
        pltpu.VMEM((CH, TW), f32),
        pltpu.VMEM((CH, H), f32),
        pltpu.VMEM((CH, H), f32),
        pltpu.VMEM((CH, 16), f32),
        pltpu.VMEM((CH, 16), f32),
        pltpu.SemaphoreType.DMA,
        pltpu.SemaphoreType.DMA,
        pltpu.SemaphoreType.DMA,
        pltpu.SemaphoreType.DMA,
    ]
    BE = 3200
    eb1r = eb1.reshape(1, H)
    eb2r = eb2.reshape(1, H)
    pb1r = pb1.reshape(1, H)
    pw2r = pW2.reshape(1, H)
    mijH = []
    wvH = []
    for h in range(2):
        gsum, rel = pl.kernel(
            _gather_sc,
            out_type=(jax.ShapeDtypeStruct((E2, H), f32),
                      jax.ShapeDtypeStruct((E2, 16), f32)),
            mesh=mesh,
            compiler_params=sc_params,
            scratch_types=gather_scratch,
        )(TA, TB, srcH[h], dstH[h])

        mij, wv = pl.pallas_call(
            _edge_body,
            grid=(E2 // BE,),
            in_specs=[
                pl.BlockSpec((BE, H), lambda i: (i, 0)),
                pl.BlockSpec((BE, 16), lambda i: (i, 0)),
                pl.BlockSpec((16, BE), lambda i: (0, i)),
                pl.BlockSpec((1, H), lambda i: (0, 0)),
                pl.BlockSpec((1, H), lambda i: (0, 0)),
                pl.BlockSpec((16, H), lambda i: (0, 0)),
                pl.BlockSpec((H, H), lambda i: (0, 0)),
                pl.BlockSpec((1, H), lambda i: (0, 0)),
                pl.BlockSpec((H, H), lambda i: (0, 0)),
                pl.BlockSpec((1, H), lambda i: (0, 0)),
                pl.BlockSpec((1, H), lambda i: (0, 0)),
            ],
            out_specs=[
                pl.BlockSpec((BE, H), lambda i: (i, 0)),
                pl.BlockSpec((BE, 16), lambda i: (i, 0)),
            ],
            out_shape=[
                jax.ShapeDtypeStruct((E2, H), f32),
                jax.ShapeDtypeStruct((E2, 16), f32),
            ],
        )(gsum, rel, eaTH[h], eb1r, wr, We, eW2, eb2r, pW1, pb1r, pw2r)
        mijH.append(mij)
        wvH.append(wv)

    scatter_scratch = [
        pltpu.VMEM((nch, CH), jnp.int32),
        pltpu.VMEM((nch, CH), jnp.int32),
        pltpu.VMEM((CH, H), f32),
        pltpu.VMEM((CH, H), f32),
        pltpu.VMEM((CH, 16), f32),
        pltpu.VMEM((CH, 16), f32),
        pltpu.VMEM_SHARED((N, H), f32),
        pltpu.VMEM_SHARED((N, 16), f32),
        pltpu.SemaphoreType.DMA,
        pltpu.SemaphoreType.DMA,
        pltpu.SemaphoreType.DMA,
        pltpu.SemaphoreType.DMA,
    ]
    mi2, pa2 = pl.kernel(
        _scatter_sc,
        out_type=(jax.ShapeDtypeStruct((NC, N, H), f32),
                  jax.ShapeDtypeStruct((NC, N, 16), f32)),
        mesh=mesh,
        compiler_params=sc_params,
        scratch_types=scatter_scratch,
    )(mijH[0], mijH[1], wvH[0], wvH[1], dstH[0], dstH[1])

    fW1a = fW1[:D]
    fW1b = fW1[D:]
    fb1r = fb1.reshape(1, H)
    fb2r = fb2.reshape(1, D)
    feats, posp = pl.pallas_call(
        _final_body,
        grid=(N // BN,),
        in_specs=[
            pl.BlockSpec((BN, D), lambda i: (i, 0)),
            pl.BlockSpec((BN, 16), lambda i: (i, 0)),
            pl.BlockSpec((NC, BN, H), lambda i: (0, i, 0)),
            pl.BlockSpec((NC, BN, 16), lambda i: (0, i, 0)),
            pl.BlockSpec((D, H), lambda i: (0, 0)),
            pl.BlockSpec((H, H), lambda i: (0, 0)),
            pl.BlockSpec((1, H), lambda i: (0, 0)),
            pl.BlockSpec((H, D), lambda i: (0, 0)),
            pl.BlockSpec((1, D), lambda i: (0, 0)),
        ],
        out_specs=[
            pl.BlockSpec((BN, D), lambda i: (i, 0)),
            pl.BlockSpec((BN, 16), lambda i: (i, 0)),
        ],
        out_shape=[
            jax.ShapeDtypeStruct((N, D), f32),
            jax.ShapeDtypeStruct((N, 16), f32),
        ],
    )(node_feats, P, mi2, pa2, fW1a, fW1b, fb1r, fW2, fb2r)

    return (feats, posp[:, :3])
